# Initial kernel scaffold; baseline (speedup 1.0000x reference)
import jax
import jax.numpy as jnp
from jax.experimental import pallas as pl


def kernel(y_true, mask, centers):
    raise NotImplementedError("write your pallas kernel here")


# speedup vs baseline: 1.0914x; 1.0914x over previous
import jax
import jax.numpy as jnp
from jax.experimental import pallas as pl

NUM_CENTERS = 512
CODE_DIM = 32
ROWS = 1024


def _assign_oneh_body(x_ref, m_ref, c_ref, o_ref):
    x = x_ref[...] * (1.0 - m_ref[...])
    c = c_ref[...]
    x2 = jnp.sum(x * x, axis=1, keepdims=True)
    c2 = jnp.sum(c * c, axis=1)[None, :]
    xc = jax.lax.dot_general(
        x, c, (((1,), (1,)), ((), ())), preferred_element_type=jnp.float32)
    d = x2 - 2.0 * xc + c2
    dmin = jnp.min(d, axis=1, keepdims=True)
    iota = jax.lax.broadcasted_iota(jnp.int32, d.shape, 1)
    masked = jnp.where(d == dmin, iota, NUM_CENTERS)
    idx = jnp.min(masked, axis=1)[:, None]
    o_ref[...] = (iota == idx).astype(jnp.float32)


def kernel(y_true, mask, centers):
    B, T, n, d = y_true.shape
    N = B * T * n
    x = y_true.reshape(N, d)
    m = mask.reshape(N, d)
    out = pl.pallas_call(
        _assign_oneh_body,
        grid=(N // ROWS,),
        in_specs=[
            pl.BlockSpec((ROWS, d), lambda i: (i, 0)),
            pl.BlockSpec((ROWS, d), lambda i: (i, 0)),
            pl.BlockSpec((NUM_CENTERS, d), lambda i: (0, 0)),
        ],
        out_specs=pl.BlockSpec((ROWS, NUM_CENTERS), lambda i: (i, 0)),
        out_shape=jax.ShapeDtypeStruct((N, NUM_CENTERS), jnp.float32),
    )(x, m, centers)
    return out.reshape(B, T, n, NUM_CENTERS)
